# baseline (speedup 1.0000x reference)
import functools

import jax
import jax.numpy as jnp
from jax import lax
from jax.experimental import pallas as pl
from jax.experimental.pallas import tpu as pltpu
from jax.experimental.pallas import tpu_sc as plsc

_VOCAB = 1000000
_EMB = 64
_BATCH = 4096
_SEQ = 200
_OUT = 16

_NC = 2
_NS = 16
_NW = _NC * _NS
_BPW = _BATCH // _NW

_CHUNKS = ((0, 128), (128, 72))


def _gather_pool_body(x_hbm, table_hbm, out_hbm, idx_v, rows_v, acc_v, sem):
  wid = lax.axis_index("s") * _NC + lax.axis_index("c")
  base = pl.multiple_of(wid * _BPW, _BPW)

  pltpu.sync_copy(x_hbm.at[pl.ds(pl.multiple_of(base * _SEQ, 8), _BPW * _SEQ)],
                  idx_v)

  def do_sample(s, _):
    off = pl.multiple_of(s * _SEQ, 8)
    cps = []
    for (o, n) in _CHUNKS:
      cps.append(pltpu.async_copy(
          table_hbm.at[idx_v.at[pl.ds(off + o, n)]],
          rows_v.at[pl.ds(o, n)], sem))
    for cp in cps:
      cp.wait()

    def rb(i, accs):
      a = list(accs)
      for j in range(8):
        r = i * 8 + j
        for c in range(4):
          a[c] = a[c] + rows_v[r, pl.ds(c * 16, 16)]
      return tuple(a)

    zero = jnp.zeros((16,), jnp.float32)
    accs = lax.fori_loop(0, _SEQ // 8, rb, (zero, zero, zero, zero))
    for c in range(4):
      acc_v[s, pl.ds(c * 16, 16)] = accs[c]
    return 0

  lax.fori_loop(0, _BPW, do_sample, 0)

  pltpu.sync_copy(acc_v, out_hbm.at[pl.ds(base, _BPW)])


_gather_pool = functools.partial(
    pl.kernel,
    out_type=jax.ShapeDtypeStruct((_BATCH, _EMB), jnp.float32),
    mesh=plsc.VectorSubcoreMesh(core_axis_name="c", subcore_axis_name="s"),
    compiler_params=pltpu.CompilerParams(use_tc_tiling_on_sc=False),
    scratch_types=[
        pltpu.VMEM((_BPW * _SEQ,), jnp.int32),
        pltpu.VMEM((_SEQ, _EMB), jnp.float32),
        pltpu.VMEM((_BPW, _EMB), jnp.float32),
        pltpu.SemaphoreType.DMA,
    ],
)(_gather_pool_body)


def _mlp_body(rep_ref, len_ref, fcwt_ref, fcb_ref, clfwt_ref, clfb_ref,
              out_ref):
  r = rep_ref[...] / len_ref[...]
  h = jnp.maximum(
      jnp.dot(r, fcwt_ref[...], preferred_element_type=jnp.float32)
      + fcb_ref[...], 0.0)
  out_ref[...] = (
      jnp.dot(h, clfwt_ref[...], preferred_element_type=jnp.float32)
      + clfb_ref[...])


def _mlp(rep, len_f, fcwt, fcb2, clfwt, clfb2):
  return pl.pallas_call(
      _mlp_body,
      out_shape=jax.ShapeDtypeStruct((_BATCH, _OUT), jnp.float32),
  )(rep, len_f, fcwt, fcb2, clfwt, clfb2)


def kernel(x, lengths, table, fc_w, fc_b, clf_w, clf_b):
  reps = _gather_pool(x.reshape(-1), table)
  len_f = lengths.astype(jnp.float32).reshape(_BATCH, 1)
  return _mlp(reps, len_f, fc_w.T, fc_b.reshape(1, _OUT), clf_w.T,
              clf_b.reshape(1, _OUT))


# speedup vs baseline: 1.1966x; 1.1966x over previous
import functools

import jax
import jax.numpy as jnp
from jax import lax
from jax.experimental import pallas as pl
from jax.experimental.pallas import tpu as pltpu
from jax.experimental.pallas import tpu_sc as plsc

_VOCAB = 1000000
_EMB = 64
_BATCH = 4096
_SEQ = 200
_OUT = 16

_NC = 2
_NS = 16
_NW = _NC * _NS
_BPW = _BATCH // _NW

_CHUNKS = ((0, 128), (128, 72))


_NBUF = 4


def _gather_pool_body(x_hbm, table_hbm, out_hbm, idx_v, rows_v, acc_v, sems):
  wid = lax.axis_index("s") * _NC + lax.axis_index("c")
  base = pl.multiple_of(wid * _BPW, _BPW)

  pltpu.sync_copy(x_hbm.at[pl.ds(pl.multiple_of(base * _SEQ, 8), _BPW * _SEQ)],
                  idx_v)

  def issue(s, b):
    off = pl.multiple_of(s * _SEQ, 8)
    for (o, n) in _CHUNKS:
      pltpu.async_copy(
          table_hbm.at[idx_v.at[pl.ds(off + o, n)]],
          rows_v.at[b, pl.ds(o, n)], sems.at[b])

  def wait(b):
    pltpu.make_async_copy(
        table_hbm.at[pl.ds(0, _SEQ)], rows_v.at[b], sems.at[b]).wait()

  for b in range(_NBUF):
    issue(b, b)

  def do_group(g, _):
    for b in range(_NBUF):
      s = g * _NBUF + b
      wait(b)

      def rb(i, accs):
        a = list(accs)
        for j in range(8):
          r = i * 8 + j
          for c in range(4):
            a[c] = a[c] + rows_v[b, r, pl.ds(c * 16, 16)]
        return tuple(a)

      zero = jnp.zeros((16,), jnp.float32)
      accs = lax.fori_loop(0, _SEQ // 8, rb, (zero, zero, zero, zero))
      for c in range(4):
        acc_v[s, pl.ds(c * 16, 16)] = accs[c]

      @pl.when(s + _NBUF < _BPW)
      def _():
        issue(s + _NBUF, b)
    return 0

  lax.fori_loop(0, _BPW // _NBUF, do_group, 0)

  pltpu.sync_copy(acc_v, out_hbm.at[pl.ds(base, _BPW)])


_gather_pool = functools.partial(
    pl.kernel,
    out_type=jax.ShapeDtypeStruct((_BATCH, _EMB), jnp.float32),
    mesh=plsc.VectorSubcoreMesh(core_axis_name="c", subcore_axis_name="s"),
    compiler_params=pltpu.CompilerParams(use_tc_tiling_on_sc=False),
    scratch_types=[
        pltpu.VMEM((_BPW * _SEQ,), jnp.int32),
        pltpu.VMEM((_NBUF, _SEQ, _EMB), jnp.float32),
        pltpu.VMEM((_BPW, _EMB), jnp.float32),
        pltpu.SemaphoreType.DMA((_NBUF,)),
    ],
)(_gather_pool_body)


def _mlp_body(rep_ref, len_ref, fcwt_ref, fcb_ref, clfwt_ref, clfb_ref,
              out_ref):
  r = rep_ref[...] / len_ref[...]
  h = jnp.maximum(
      jnp.dot(r, fcwt_ref[...], preferred_element_type=jnp.float32)
      + fcb_ref[...], 0.0)
  out_ref[...] = (
      jnp.dot(h, clfwt_ref[...], preferred_element_type=jnp.float32)
      + clfb_ref[...])


def _mlp(rep, len_f, fcwt, fcb2, clfwt, clfb2):
  return pl.pallas_call(
      _mlp_body,
      out_shape=jax.ShapeDtypeStruct((_BATCH, _OUT), jnp.float32),
  )(rep, len_f, fcwt, fcb2, clfwt, clfb2)


def kernel(x, lengths, table, fc_w, fc_b, clf_w, clf_b):
  reps = _gather_pool(x.reshape(-1), table)
  len_f = lengths.astype(jnp.float32).reshape(_BATCH, 1)
  return _mlp(reps, len_f, fc_w.T, fc_b.reshape(1, _OUT), clf_w.T,
              clf_b.reshape(1, _OUT))


# speedup vs baseline: 1.1973x; 1.0005x over previous
import functools

import jax
import jax.numpy as jnp
from jax import lax
from jax.experimental import pallas as pl
from jax.experimental.pallas import tpu as pltpu
from jax.experimental.pallas import tpu_sc as plsc

_VOCAB = 1000000
_EMB = 64
_BATCH = 4096
_SEQ = 200
_OUT = 16

_NC = 2
_NS = 16
_NW = _NC * _NS
_BPW = _BATCH // _NW

_CHUNKS = ((0, 128), (128, 72))


_NBUF = 4


def _gather_pool_body(x_hbm, table_hbm, out_hbm, idx_v, rows_v, acc_v, sems):
  wid = lax.axis_index("s") * _NC + lax.axis_index("c")
  base = pl.multiple_of(wid * _BPW, _BPW)

  pltpu.sync_copy(x_hbm.at[pl.ds(base, _BPW)], idx_v)

  def issue(s, b):
    for (o, n) in _CHUNKS:
      pltpu.async_copy(
          table_hbm.at[idx_v.at[s, pl.ds(o, n)]],
          rows_v.at[b, pl.ds(o, n)], sems.at[b])

  def wait(b):
    pltpu.make_async_copy(
        table_hbm.at[pl.ds(0, _SEQ)], rows_v.at[b], sems.at[b]).wait()

  for b in range(_NBUF):
    issue(b, b)

  def do_group(g, _):
    for b in range(_NBUF):
      s = g * _NBUF + b
      wait(b)

      def rb(i, accs):
        a = list(accs)
        for j in range(8):
          r = i * 8 + j
          for c in range(4):
            a[c] = a[c] + rows_v[b, r, pl.ds(c * 16, 16)]
        return tuple(a)

      zero = jnp.zeros((16,), jnp.float32)
      accs = lax.fori_loop(0, _SEQ // 8, rb, (zero, zero, zero, zero))
      for c in range(4):
        acc_v[s, pl.ds(c * 16, 16)] = accs[c]

      @pl.when(s + _NBUF < _BPW)
      def _():
        issue(s + _NBUF, b)
    return 0

  lax.fori_loop(0, _BPW // _NBUF, do_group, 0)

  pltpu.sync_copy(acc_v, out_hbm.at[pl.ds(base, _BPW)])


_gather_pool = functools.partial(
    pl.kernel,
    out_type=jax.ShapeDtypeStruct((_BATCH, _EMB), jnp.float32),
    mesh=plsc.VectorSubcoreMesh(core_axis_name="c", subcore_axis_name="s"),
    compiler_params=pltpu.CompilerParams(use_tc_tiling_on_sc=False),
    scratch_types=[
        pltpu.VMEM((_BPW, _SEQ), jnp.int32),
        pltpu.VMEM((_NBUF, _SEQ, _EMB), jnp.float32),
        pltpu.VMEM((_BPW, _EMB), jnp.float32),
        pltpu.SemaphoreType.DMA((_NBUF,)),
    ],
)(_gather_pool_body)


def _mlp_body(rep_ref, len_ref, fcwt_ref, fcb_ref, clfwt_ref, clfb_ref,
              out_ref):
  r = rep_ref[...] / len_ref[...]
  h = jnp.maximum(
      jnp.dot(r, fcwt_ref[...], preferred_element_type=jnp.float32)
      + fcb_ref[...], 0.0)
  out_ref[...] = (
      jnp.dot(h, clfwt_ref[...], preferred_element_type=jnp.float32)
      + clfb_ref[...])


def _mlp(rep, len_f, fcwt, fcb2, clfwt, clfb2):
  return pl.pallas_call(
      _mlp_body,
      out_shape=jax.ShapeDtypeStruct((_BATCH, _OUT), jnp.float32),
  )(rep, len_f, fcwt, fcb2, clfwt, clfb2)


def kernel(x, lengths, table, fc_w, fc_b, clf_w, clf_b):
  reps = _gather_pool(x, table)
  len_f = lengths.astype(jnp.float32).reshape(_BATCH, 1)
  return _mlp(reps, len_f, fc_w.T, fc_b.reshape(1, _OUT), clf_w.T,
              clf_b.reshape(1, _OUT))
